# baseline (speedup 1.0000x reference)
import functools

import jax
import jax.numpy as jnp
from jax.experimental import pallas as pl
from jax.experimental.pallas import tpu as pltpu

LANE = 128
POOL = 32
KSZ = 32


def _toeplitz(w):
    cout, cin, k = w.shape
    s = jnp.arange(2 * POOL)[None, :]
    r = jnp.arange(POOL)[:, None]
    d = s - r
    mask = (d >= 0) & (d < k)
    g = w[:, :, jnp.clip(d, 0, k - 1)]
    g = jnp.where(mask[None, None], g, 0.0)
    g = g.transpose(0, 2, 1, 3)
    return g.reshape(cout * POOL, cin * 2 * POOL)


def _stage0_kernel(b0_ref, ws_ref, wh_ref, xh_ref, xl_ref, y_ref, *, mcol, j0, bb):
    mpad = y_ref.shape[-1]
    for b in range(bb):
        xth = jnp.transpose(xh_ref[b], (1, 0))
        xtl = jnp.transpose(xl_ref[b], (1, 0))
        for a in range(4):
            if a < 3:
                xeh = xth[POOL * a:POOL * a + 2 * POOL, :mcol]
                xel = xtl[POOL * a:POOL * a + 2 * POOL, :mcol]
            else:
                xeh = jnp.concatenate(
                    [xth[3 * POOL:, :mcol], xth[:POOL, 1:mcol + 1]], axis=0)
                xel = jnp.concatenate(
                    [xtl[3 * POOL:, :mcol], xtl[:POOL, 1:mcol + 1]], axis=0)
            ch = jnp.dot(ws_ref[...], xeh, preferred_element_type=jnp.float32)
            cl = jnp.dot(wh_ref[...], xel, preferred_element_type=jnp.float32)
            c = ch[:2 * POOL] + ch[2 * POOL:] + cl
            bound = (j0 - 1 - a) // 4 + 1
            mask = jax.lax.broadcasted_iota(jnp.int32, (1, mpad), 1) < bound
            for co in range(2):
                pooled = jnp.max(c[POOL * co:POOL * (co + 1), :], axis=0,
                                 keepdims=True)
                row = jnp.maximum(pooled + b0_ref[co], 0.0)
                row = jnp.pad(row, ((0, 0), (0, mpad - mcol)))
                y_ref[b, 4 * co + a:4 * co + a + 1, :] = jnp.where(mask, row, 0.0)


def _stage1_kernel(b1_ref, w_ref, xe_ref, wl_ref, bl_ref, out_ref, *, bb):
    for b in range(bb):
        c = jnp.dot(w_ref[...], xe_ref[b], preferred_element_type=jnp.float32,
                    precision=jax.lax.Precision.HIGHEST)
        acc = jnp.zeros((8, LANE), dtype=jnp.float32)
        for co in range(4):
            pooled = jnp.max(c[POOL * co:POOL * (co + 1), :], axis=0, keepdims=True)
            z = jnp.maximum(pooled + b1_ref[co], 0.0)
            lhs = jnp.broadcast_to(z, (8, LANE))
            acc = acc + jnp.dot(lhs, wl_ref[co],
                                preferred_element_type=jnp.float32,
                                precision=jax.lax.Precision.HIGHEST)
        out_ref[b] = acc + bl_ref[...]


def _phase_major(x, j_out, jpad):
    b, cin, l = x.shape
    need = POOL * (j_out + 2)
    xpad = jnp.pad(x, ((0, 0), (0, 0), (0, max(0, need - l))))
    a = xpad[..., :POOL * j_out].reshape(b, cin, j_out, POOL).transpose(0, 1, 3, 2)
    c = xpad[..., POOL:POOL * (j_out + 1)].reshape(b, cin, j_out, POOL).transpose(0, 1, 3, 2)
    xe = jnp.concatenate([a, c], axis=2)
    xe = jnp.pad(xe, ((0, 0), (0, 0), (0, 0), (0, jpad - j_out)))
    return xe.reshape(b, cin * 2 * POOL, jpad)


def kernel(x, w0, b0, w1, b1, wl, bl):
    B, Cin, L = x.shape
    O = wl.shape[0]
    j0 = (L - KSZ + 1) // POOL
    j0pad = pl.cdiv(j0 + 2, LANE) * LANE
    j1 = (j0 - KSZ + 1) // POOL

    mcol = j0pad // 4
    nrow = mcol - 4
    xp = jnp.pad(x.reshape(B, L), ((0, 0), (0, nrow * LANE - L)))
    xr = xp.reshape(B, nrow, LANE)
    xh = xr.astype(jnp.bfloat16)
    xl = (xr - xh.astype(jnp.float32)).astype(jnp.bfloat16)

    t0 = _toeplitz(w0)
    t0h = t0.astype(jnp.bfloat16)
    t0l = (t0 - t0h.astype(jnp.float32)).astype(jnp.bfloat16)
    t0s = jnp.concatenate([t0h, t0l], axis=0)

    BB0 = 2 if B % 2 == 0 else 1
    grid0 = pltpu.PrefetchScalarGridSpec(
        num_scalar_prefetch=1,
        grid=(B // BB0,),
        in_specs=[
            pl.BlockSpec((4 * POOL, 2 * POOL), lambda bi, sm: (0, 0)),
            pl.BlockSpec((2 * POOL, 2 * POOL), lambda bi, sm: (0, 0)),
            pl.BlockSpec((BB0, nrow, LANE), lambda bi, sm: (bi, 0, 0)),
            pl.BlockSpec((BB0, nrow, LANE), lambda bi, sm: (bi, 0, 0)),
        ],
        out_specs=pl.BlockSpec((BB0, 8, mcol), lambda bi, sm: (bi, 0, 0)),
    )
    y0am = pl.pallas_call(
        functools.partial(_stage0_kernel, mcol=nrow - 1, j0=j0, bb=BB0),
        out_shape=jax.ShapeDtypeStruct((B, 8, mcol), jnp.float32),
        grid_spec=grid0,
        compiler_params=pltpu.CompilerParams(dimension_semantics=("parallel",)),
    )(b0, t0s, t0h, xh, xl)

    y0 = y0am.reshape(B, 2, 4, mcol).transpose(0, 1, 3, 2).reshape(B, 2, 4 * mcol)
    xe1 = _phase_major(y0, j1, LANE)

    t1 = _toeplitz(w1)

    wl_r = wl.reshape(O, 4, j1).transpose(1, 2, 0)
    wl_r = jnp.pad(wl_r, ((0, 0), (0, LANE - j1), (0, LANE - O)))
    bl_p = jnp.pad(bl, (0, LANE - O)).reshape(1, LANE)

    BB1 = 8 if B % 8 == 0 else 1
    grid1 = pltpu.PrefetchScalarGridSpec(
        num_scalar_prefetch=1,
        grid=(B // BB1,),
        in_specs=[
            pl.BlockSpec((4 * POOL, 4 * POOL), lambda bi, sm: (0, 0)),
            pl.BlockSpec((BB1, LANE, LANE), lambda bi, sm: (bi, 0, 0)),
            pl.BlockSpec((4, LANE, LANE), lambda bi, sm: (0, 0, 0)),
            pl.BlockSpec((1, LANE), lambda bi, sm: (0, 0)),
        ],
        out_specs=pl.BlockSpec((BB1, 8, LANE), lambda bi, sm: (bi, 0, 0)),
    )
    out = pl.pallas_call(
        functools.partial(_stage1_kernel, bb=BB1),
        out_shape=jax.ShapeDtypeStruct((B, 8, LANE), jnp.float32),
        grid_spec=grid1,
        compiler_params=pltpu.CompilerParams(dimension_semantics=("parallel",)),
    )(b1, t1, xe1, wl_r, bl_p)
    return out[:, 0, :O]


# speedup vs baseline: 1.4273x; 1.4273x over previous
import functools

import jax
import jax.numpy as jnp
from jax.experimental import pallas as pl
from jax.experimental.pallas import tpu as pltpu

LANE = 128
POOL = 32
KSZ = 32


def _toeplitz(w):
    cout, cin, k = w.shape
    s = jnp.arange(2 * POOL)[None, :]
    r = jnp.arange(POOL)[:, None]
    d = s - r
    mask = (d >= 0) & (d < k)
    g = w[:, :, jnp.clip(d, 0, k - 1)]
    g = jnp.where(mask[None, None], g, 0.0)
    g = g.transpose(0, 2, 1, 3)
    return g.reshape(cout * POOL, cin * 2 * POOL)


def _stage0_kernel(b0_ref, ws_ref, wh_ref, x_ref, y_ref, *, mcol, j0, bb):
    mpad = y_ref.shape[-1]
    for b in range(bb):
        xt = jnp.transpose(x_ref[b], (1, 0))
        xth = xt.astype(jnp.bfloat16)
        xtl = (xt - xth.astype(jnp.float32)).astype(jnp.bfloat16)
        for a in range(4):
            if a < 3:
                xeh = xth[POOL * a:POOL * a + 2 * POOL, :mcol]
                xel = xtl[POOL * a:POOL * a + 2 * POOL, :mcol]
            else:
                xeh = jnp.concatenate(
                    [xth[3 * POOL:, :mcol], xth[:POOL, 1:mcol + 1]], axis=0)
                xel = jnp.concatenate(
                    [xtl[3 * POOL:, :mcol], xtl[:POOL, 1:mcol + 1]], axis=0)
            ch = jnp.dot(ws_ref[...], xeh, preferred_element_type=jnp.float32)
            cl = jnp.dot(wh_ref[...], xel, preferred_element_type=jnp.float32)
            c = ch[:2 * POOL] + ch[2 * POOL:] + cl
            bound = (j0 - 1 - a) // 4 + 1
            mask = jax.lax.broadcasted_iota(jnp.int32, (1, mpad), 1) < bound
            for co in range(2):
                pooled = jnp.max(c[POOL * co:POOL * (co + 1), :], axis=0,
                                 keepdims=True)
                row = jnp.maximum(pooled + b0_ref[co], 0.0)
                row = jnp.pad(row, ((0, 0), (0, mpad - mcol)))
                y_ref[b, 4 * co + a:4 * co + a + 1, :] = jnp.where(mask, row, 0.0)


def _stage1_kernel(b1_ref, ws_ref, wh_ref, xe_ref, wlh_ref, wll_ref, bl_ref,
                   out_ref, *, bb):
    xe = xe_ref[0]
    xeh = xe.astype(jnp.bfloat16)
    xel = (xe - xeh.astype(jnp.float32)).astype(jnp.bfloat16)
    ch = jnp.dot(ws_ref[...], xeh, preferred_element_type=jnp.float32)
    cl = jnp.dot(wh_ref[...], xel, preferred_element_type=jnp.float32)
    c = ch[:4 * POOL] + ch[4 * POOL:] + cl
    zs = []
    for co in range(4):
        pooled = jnp.max(c[POOL * co:POOL * (co + 1), :], axis=0, keepdims=True)
        zs.append(jnp.maximum(pooled + b1_ref[co], 0.0))
    acc = jnp.zeros((bb, LANE), dtype=jnp.float32)
    for co in range(4):
        z = jnp.concatenate(
            [zs[co][:, LANE * b:LANE * (b + 1)] for b in range(bb)], axis=0)
        zh = z.astype(jnp.bfloat16)
        zl = (z - zh.astype(jnp.float32)).astype(jnp.bfloat16)
        d = jnp.dot(jnp.concatenate([zh, zl], axis=0), wlh_ref[co],
                    preferred_element_type=jnp.float32)
        acc = acc + d[:bb] + d[bb:] + jnp.dot(zh, wll_ref[co],
                                              preferred_element_type=jnp.float32)
    out_ref[...] = acc + bl_ref[...]


def _phase_major(x, j_out, jpad):
    b, cin, l = x.shape
    need = POOL * (j_out + 2)
    xpad = jnp.pad(x, ((0, 0), (0, 0), (0, max(0, need - l))))
    a = xpad[..., :POOL * j_out].reshape(b, cin, j_out, POOL).transpose(0, 1, 3, 2)
    c = xpad[..., POOL:POOL * (j_out + 1)].reshape(b, cin, j_out, POOL).transpose(0, 1, 3, 2)
    xe = jnp.concatenate([a, c], axis=2)
    xe = jnp.pad(xe, ((0, 0), (0, 0), (0, 0), (0, jpad - j_out)))
    return xe.reshape(b, cin * 2 * POOL, jpad)


def kernel(x, w0, b0, w1, b1, wl, bl):
    B, Cin, L = x.shape
    O = wl.shape[0]
    j0 = (L - KSZ + 1) // POOL
    j0pad = pl.cdiv(j0 + 2, LANE) * LANE
    j1 = (j0 - KSZ + 1) // POOL

    mcol = j0pad // 4
    nrow = mcol - 4
    xp = jnp.pad(x.reshape(B, L), ((0, 0), (0, nrow * LANE - L)))
    xr = xp.reshape(B, nrow, LANE)

    t0 = _toeplitz(w0)
    t0h = t0.astype(jnp.bfloat16)
    t0l = (t0 - t0h.astype(jnp.float32)).astype(jnp.bfloat16)
    t0s = jnp.concatenate([t0h, t0l], axis=0)

    BB0 = 2 if B % 2 == 0 else 1
    grid0 = pltpu.PrefetchScalarGridSpec(
        num_scalar_prefetch=1,
        grid=(B // BB0,),
        in_specs=[
            pl.BlockSpec((4 * POOL, 2 * POOL), lambda bi, sm: (0, 0)),
            pl.BlockSpec((2 * POOL, 2 * POOL), lambda bi, sm: (0, 0)),
            pl.BlockSpec((BB0, nrow, LANE), lambda bi, sm: (bi, 0, 0)),
        ],
        out_specs=pl.BlockSpec((BB0, 8, mcol), lambda bi, sm: (bi, 0, 0)),
    )
    y0am = pl.pallas_call(
        functools.partial(_stage0_kernel, mcol=nrow - 1, j0=j0, bb=BB0),
        out_shape=jax.ShapeDtypeStruct((B, 8, mcol), jnp.float32),
        grid_spec=grid0,
        compiler_params=pltpu.CompilerParams(dimension_semantics=("parallel",)),
    )(b0, t0s, t0h, xr)

    y0 = y0am.reshape(B, 2, 4, mcol).transpose(0, 1, 3, 2).reshape(B, 2, 4 * mcol)
    xe1 = _phase_major(y0, j1, LANE)

    t1 = _toeplitz(w1)
    t1h = t1.astype(jnp.bfloat16)
    t1l = (t1 - t1h.astype(jnp.float32)).astype(jnp.bfloat16)
    t1s = jnp.concatenate([t1h, t1l], axis=0)

    wl_r = wl.reshape(O, 4, j1).transpose(1, 2, 0)
    wl_r = jnp.pad(wl_r, ((0, 0), (0, LANE - j1), (0, LANE - O)))
    wlh = wl_r.astype(jnp.bfloat16)
    wll = (wl_r - wlh.astype(jnp.float32)).astype(jnp.bfloat16)
    bl_p = jnp.pad(bl, (0, LANE - O)).reshape(1, LANE)

    BB1 = 8 if B % 8 == 0 else (4 if B % 4 == 0 else 1)
    xe1g = (xe1.reshape(B // BB1, BB1, LANE, LANE)
            .transpose(0, 2, 1, 3).reshape(B // BB1, LANE, BB1 * LANE))
    grid1 = pltpu.PrefetchScalarGridSpec(
        num_scalar_prefetch=1,
        grid=(B // BB1,),
        in_specs=[
            pl.BlockSpec((8 * POOL, 4 * POOL), lambda bi, sm: (0, 0)),
            pl.BlockSpec((4 * POOL, 4 * POOL), lambda bi, sm: (0, 0)),
            pl.BlockSpec((1, LANE, BB1 * LANE), lambda bi, sm: (bi, 0, 0)),
            pl.BlockSpec((4, LANE, LANE), lambda bi, sm: (0, 0, 0)),
            pl.BlockSpec((4, LANE, LANE), lambda bi, sm: (0, 0, 0)),
            pl.BlockSpec((1, LANE), lambda bi, sm: (0, 0)),
        ],
        out_specs=pl.BlockSpec((BB1, LANE), lambda bi, sm: (bi, 0)),
    )
    out = pl.pallas_call(
        functools.partial(_stage1_kernel, bb=BB1),
        out_shape=jax.ShapeDtypeStruct((B, LANE), jnp.float32),
        grid_spec=grid1,
        compiler_params=pltpu.CompilerParams(dimension_semantics=("parallel",)),
    )(b1, t1s, t1h, xe1g, wlh, wll, bl_p)
    return out[:, :O]


# speedup vs baseline: 1.5301x; 1.0720x over previous
import functools

import jax
import jax.numpy as jnp
from jax.experimental import pallas as pl
from jax.experimental.pallas import tpu as pltpu

LANE = 128
POOL = 32
KSZ = 32


def _toeplitz(w):
    cout, cin, k = w.shape
    s = jnp.arange(2 * POOL)[None, :]
    r = jnp.arange(POOL)[:, None]
    d = s - r
    mask = (d >= 0) & (d < k)
    g = w[:, :, jnp.clip(d, 0, k - 1)]
    g = jnp.where(mask[None, None], g, 0.0)
    g = g.transpose(0, 2, 1, 3)
    return g.reshape(cout * POOL, cin * 2 * POOL)


def _stage0_kernel(b0_ref, ws_ref, wh_ref, x_ref, y_ref, *, mcol, j0, bb):
    mpad = y_ref.shape[-1]
    for b in range(bb):
        xt = jnp.transpose(x_ref[b], (1, 0))
        xth = xt.astype(jnp.bfloat16)
        xtl = (xt - xth.astype(jnp.float32)).astype(jnp.bfloat16)
        for a in range(4):
            if a < 3:
                xeh = xth[POOL * a:POOL * a + 2 * POOL, :mcol]
                xel = xtl[POOL * a:POOL * a + 2 * POOL, :mcol]
            else:
                xeh = jnp.concatenate(
                    [xth[3 * POOL:, :mcol], xth[:POOL, 1:mcol + 1]], axis=0)
                xel = jnp.concatenate(
                    [xtl[3 * POOL:, :mcol], xtl[:POOL, 1:mcol + 1]], axis=0)
            ch = jnp.dot(ws_ref[...], xeh, preferred_element_type=jnp.float32)
            cl = jnp.dot(wh_ref[...], xel, preferred_element_type=jnp.float32)
            c = ch[:2 * POOL] + ch[2 * POOL:] + cl
            bound = (j0 - 1 - a) // 4 + 1
            mask = jax.lax.broadcasted_iota(jnp.int32, (1, mpad), 1) < bound
            pooled2 = jnp.max(c.reshape(2, POOL, -1), axis=1)
            for co in range(2):
                row = jnp.maximum(pooled2[co:co + 1] + b0_ref[co], 0.0)
                row = jnp.pad(row, ((0, 0), (0, mpad - mcol)))
                y_ref[b, 4 * co + a:4 * co + a + 1, :] = jnp.where(mask, row, 0.0)


def _stage1_kernel(b1_ref, ws_ref, wh_ref, xe_ref, wlh_ref, wll_ref, bl_ref,
                   out_ref, *, bb):
    xe = xe_ref[0]
    xeh = xe.astype(jnp.bfloat16)
    xel = (xe - xeh.astype(jnp.float32)).astype(jnp.bfloat16)
    ch = jnp.dot(ws_ref[...], xeh, preferred_element_type=jnp.float32)
    cl = jnp.dot(wh_ref[...], xel, preferred_element_type=jnp.float32)
    c = ch[:4 * POOL] + ch[4 * POOL:] + cl
    zs = []
    for co in range(4):
        pooled = jnp.max(c[POOL * co:POOL * (co + 1), :], axis=0, keepdims=True)
        zs.append(jnp.maximum(pooled + b1_ref[co], 0.0))
    acc = jnp.zeros((bb, LANE), dtype=jnp.float32)
    for co in range(4):
        z = jnp.concatenate(
            [zs[co][:, LANE * b:LANE * (b + 1)] for b in range(bb)], axis=0)
        zh = z.astype(jnp.bfloat16)
        zl = (z - zh.astype(jnp.float32)).astype(jnp.bfloat16)
        d = jnp.dot(jnp.concatenate([zh, zl], axis=0), wlh_ref[co],
                    preferred_element_type=jnp.float32)
        acc = acc + d[:bb] + d[bb:] + jnp.dot(zh, wll_ref[co],
                                              preferred_element_type=jnp.float32)
    out_ref[...] = acc + bl_ref[...]


def _phase_major(x, j_out, jpad):
    b, cin, l = x.shape
    need = POOL * (j_out + 2)
    xpad = jnp.pad(x, ((0, 0), (0, 0), (0, max(0, need - l))))
    a = xpad[..., :POOL * j_out].reshape(b, cin, j_out, POOL).transpose(0, 1, 3, 2)
    c = xpad[..., POOL:POOL * (j_out + 1)].reshape(b, cin, j_out, POOL).transpose(0, 1, 3, 2)
    xe = jnp.concatenate([a, c], axis=2)
    xe = jnp.pad(xe, ((0, 0), (0, 0), (0, 0), (0, jpad - j_out)))
    return xe.reshape(b, cin * 2 * POOL, jpad)


def kernel(x, w0, b0, w1, b1, wl, bl):
    B, Cin, L = x.shape
    O = wl.shape[0]
    j0 = (L - KSZ + 1) // POOL
    j0pad = pl.cdiv(j0 + 2, LANE) * LANE
    j1 = (j0 - KSZ + 1) // POOL

    mcol = j0pad // 4
    nrow = mcol - 4
    xp = jnp.pad(x.reshape(B, L), ((0, 0), (0, nrow * LANE - L)))
    xr = xp.reshape(B, nrow, LANE)

    t0 = _toeplitz(w0)
    t0h = t0.astype(jnp.bfloat16)
    t0l = (t0 - t0h.astype(jnp.float32)).astype(jnp.bfloat16)
    t0s = jnp.concatenate([t0h, t0l], axis=0)

    BB0 = 4 if B % 4 == 0 else 1
    grid0 = pltpu.PrefetchScalarGridSpec(
        num_scalar_prefetch=1,
        grid=(B // BB0,),
        in_specs=[
            pl.BlockSpec((4 * POOL, 2 * POOL), lambda bi, sm: (0, 0)),
            pl.BlockSpec((2 * POOL, 2 * POOL), lambda bi, sm: (0, 0)),
            pl.BlockSpec((BB0, nrow, LANE), lambda bi, sm: (bi, 0, 0)),
        ],
        out_specs=pl.BlockSpec((BB0, 8, mcol), lambda bi, sm: (bi, 0, 0)),
    )
    y0am = pl.pallas_call(
        functools.partial(_stage0_kernel, mcol=nrow - 1, j0=j0, bb=BB0),
        out_shape=jax.ShapeDtypeStruct((B, 8, mcol), jnp.float32),
        grid_spec=grid0,
        compiler_params=pltpu.CompilerParams(dimension_semantics=("parallel",)),
    )(b0, t0s, t0h, xr)

    y0 = y0am.reshape(B, 2, 4, mcol).transpose(0, 1, 3, 2).reshape(B, 2, 4 * mcol)
    xe1 = _phase_major(y0, j1, LANE)

    t1 = _toeplitz(w1)
    t1h = t1.astype(jnp.bfloat16)
    t1l = (t1 - t1h.astype(jnp.float32)).astype(jnp.bfloat16)
    t1s = jnp.concatenate([t1h, t1l], axis=0)

    wl_r = wl.reshape(O, 4, j1).transpose(1, 2, 0)
    wl_r = jnp.pad(wl_r, ((0, 0), (0, LANE - j1), (0, LANE - O)))
    wlh = wl_r.astype(jnp.bfloat16)
    wll = (wl_r - wlh.astype(jnp.float32)).astype(jnp.bfloat16)
    bl_p = jnp.pad(bl, (0, LANE - O)).reshape(1, LANE)

    BB1 = 8 if B % 8 == 0 else (4 if B % 4 == 0 else 1)
    xe1g = (xe1.reshape(B // BB1, BB1, LANE, LANE)
            .transpose(0, 2, 1, 3).reshape(B // BB1, LANE, BB1 * LANE))
    grid1 = pltpu.PrefetchScalarGridSpec(
        num_scalar_prefetch=1,
        grid=(B // BB1,),
        in_specs=[
            pl.BlockSpec((8 * POOL, 4 * POOL), lambda bi, sm: (0, 0)),
            pl.BlockSpec((4 * POOL, 4 * POOL), lambda bi, sm: (0, 0)),
            pl.BlockSpec((1, LANE, BB1 * LANE), lambda bi, sm: (bi, 0, 0)),
            pl.BlockSpec((4, LANE, LANE), lambda bi, sm: (0, 0, 0)),
            pl.BlockSpec((4, LANE, LANE), lambda bi, sm: (0, 0, 0)),
            pl.BlockSpec((1, LANE), lambda bi, sm: (0, 0)),
        ],
        out_specs=pl.BlockSpec((BB1, LANE), lambda bi, sm: (bi, 0)),
    )
    out = pl.pallas_call(
        functools.partial(_stage1_kernel, bb=BB1),
        out_shape=jax.ShapeDtypeStruct((B, LANE), jnp.float32),
        grid_spec=grid1,
        compiler_params=pltpu.CompilerParams(dimension_semantics=("parallel",)),
    )(b1, t1s, t1h, xe1g, wlh, wll, bl_p)
    return out[:, :O]
